# baseline (speedup 1.0000x reference)
import functools

import jax
import jax.numpy as jnp
from jax import lax
from jax.experimental import pallas as pl
from jax.experimental.pallas import tpu as pltpu
from jax.experimental.pallas import tpu_sc as plsc

BATCH = 16384
NUM_CORES = 2
NUM_SUBCORES = 16
NUM_WORKERS = NUM_CORES * NUM_SUBCORES
ROWS_PER_WORKER = BATCH // NUM_WORKERS


def _sc_gather_all(users, items, user_table, item_table, user_cross_table,
                   item_cross_table):
  mesh = plsc.VectorSubcoreMesh(core_axis_name="c", subcore_axis_name="s")
  out_types = (
      jax.ShapeDtypeStruct((BATCH, 64), jnp.float32),
      jax.ShapeDtypeStruct((BATCH, 64), jnp.float32),
      jax.ShapeDtypeStruct((BATCH, 32), jnp.float32),
      jax.ShapeDtypeStruct((BATCH, 32), jnp.float32),
  )

  @functools.partial(
      pl.kernel,
      mesh=mesh,
      out_type=out_types,
      compiler_params=pltpu.CompilerParams(use_tc_tiling_on_sc=False),
      scratch_types=[
          pltpu.VMEM((ROWS_PER_WORKER,), jnp.int32),
          pltpu.VMEM((ROWS_PER_WORKER,), jnp.int32),
          pltpu.VMEM((ROWS_PER_WORKER, 64), jnp.float32),
          pltpu.VMEM((ROWS_PER_WORKER, 64), jnp.float32),
          pltpu.VMEM((ROWS_PER_WORKER, 32), jnp.float32),
          pltpu.VMEM((ROWS_PER_WORKER, 32), jnp.float32),
          pltpu.SemaphoreType.DMA,
      ],
  )
  def k(users_hbm, items_hbm, ut_hbm, it_hbm, uct_hbm, ict_hbm,
        ue_out, ie_out, cu_out, ci_out,
        uidx_v, iidx_v, ue_v, ie_v, cu_v, ci_v, sem):
    wid = lax.axis_index("s") * NUM_CORES + lax.axis_index("c")
    base = wid * ROWS_PER_WORKER
    sl = pl.ds(base, ROWS_PER_WORKER)
    pltpu.sync_copy(users_hbm.at[sl], uidx_v)
    pltpu.sync_copy(items_hbm.at[sl], iidx_v)
    g1 = pltpu.async_copy(ut_hbm.at[uidx_v], ue_v, sem)
    g2 = pltpu.async_copy(it_hbm.at[iidx_v], ie_v, sem)
    g3 = pltpu.async_copy(uct_hbm.at[uidx_v], cu_v, sem)
    g4 = pltpu.async_copy(ict_hbm.at[iidx_v], ci_v, sem)
    g1.wait()
    g2.wait()
    g3.wait()
    g4.wait()
    pltpu.sync_copy(ue_v, ue_out.at[sl])
    pltpu.sync_copy(ie_v, ie_out.at[sl])
    pltpu.sync_copy(cu_v, cu_out.at[sl])
    pltpu.sync_copy(ci_v, ci_out.at[sl])

  return k(users, items, user_table, item_table, user_cross_table,
           item_cross_table)


def _cross_body(cu_ref, ci_ref, o_ref):
  o_ref[...] = jnp.sum(cu_ref[...] * ci_ref[...], axis=1, keepdims=True)


def _cross_tc(cross_users, cross_items):
  return pl.pallas_call(
      _cross_body,
      out_shape=jax.ShapeDtypeStruct((BATCH, 1), jnp.float32),
  )(cross_users, cross_items)


def kernel(users, items, user_table, item_table, user_cross_table,
           item_cross_table):
  ue, ie, cu, ci = _sc_gather_all(users, items, user_table, item_table,
                                  user_cross_table, item_cross_table)
  cross = _cross_tc(cu, ci)
  return (ue, ie, cu, ci, cross)
